# baseline (speedup 1.0000x reference)
import jax
import jax.numpy as jnp
from jax import lax
from jax.experimental import pallas as pl
from jax.experimental.pallas import tpu as pltpu
from jax.experimental.pallas import tpu_sc as plsc

N_NODES = 10000
N_EDGES = 320000
D = 128
DH = D // 2

NC = 2
NS = 16
NW = NC * NS

EDGES_PER_TILE = 20480
CHUNK = 128
NCHUNK = EDGES_PER_TILE // CHUNK
N_PAD = 10240
ROWS_PER_TILE = N_PAD // NS
ZROWS = 32
DEG_W = 16


def _sc_body(h_hbm, src_hbm, dst_hbm, agg_out, deg_out, src_v, dst_v, rows,
             ones_v, zb, zbd, agg_sh, deg_sh, gsem, ssem):
    c = lax.axis_index("c")
    s = lax.axis_index("s")
    wid = c * NS + s

    cp_src = pltpu.async_copy(src_hbm.at[wid], src_v, gsem.at[0])
    cp_dst = pltpu.async_copy(dst_hbm.at[wid], dst_v, gsem.at[1])

    zeros32 = jnp.zeros((32,), jnp.bfloat16)
    zeros16 = jnp.zeros((16,), jnp.float32)
    ones16 = jnp.ones((16,), jnp.float32)

    def zrow(i, carry):
        for k in range(DH // 32):
            zb[i, pl.ds(k * 32, 32)] = zeros32
        zbd[i] = zeros16
        return carry

    lax.fori_loop(0, ZROWS, zrow, 0)

    def orow(i, carry):
        ones_v[i] = ones16
        return carry

    lax.fori_loop(0, CHUNK, orow, 0)

    base = s * ROWS_PER_TILE
    zcps = []
    for k in range(ROWS_PER_TILE // ZROWS):
        zcps.append(pltpu.async_copy(
            zb, agg_sh.at[pl.ds(base + k * ZROWS, ZROWS)], ssem.at[k % 4]))
        zcps.append(pltpu.async_copy(
            zbd, deg_sh.at[pl.ds(base + k * ZROWS, ZROWS)], ssem.at[k % 4]))
    for cp in zcps:
        cp.wait()
    cp_src.wait()
    cp_dst.wait()
    plsc.subcore_barrier()

    def gstart(j, b):
        pltpu.async_copy(h_hbm.at[src_v.at[j]], rows.at[b], gsem.at[b])

    def gwait(j, b):
        pltpu.make_async_copy(h_hbm.at[src_v.at[j]], rows.at[b],
                              gsem.at[b]).wait()

    def sstart(j, b):
        pltpu.async_copy(rows.at[b], agg_sh.at[dst_v.at[j]], ssem.at[b],
                         add=True)

    def swait(j, b):
        pltpu.make_async_copy(rows.at[b], agg_sh.at[dst_v.at[j]],
                              ssem.at[b]).wait()

    def dscat(j, deg_core):
        @pl.when(c == deg_core)
        def _():
            pltpu.sync_copy(ones_v, deg_sh.at[dst_v.at[j]], add=True)

    gstart(0, 0)
    gstart(1, 1)
    gstart(2, 2)

    def step(k4, carry):
        for b in range(4):
            j = 4 * k4 + b
            gwait(j, b)
            sstart(j, b)
            dscat(j, b % 2)
            nb = (b + 3) % 4
            if b == 0:
                @pl.when(k4 > 0)
                def _():
                    swait(j - 1, nb)
                gstart(j + 3, nb)
            else:
                @pl.when(k4 < NCHUNK // 4 - 1)
                def _():
                    swait(j - 1, nb)
                    gstart(j + 3, nb)
        return carry

    lax.fori_loop(0, NCHUNK // 4, step, 0)
    for b in range(4):
        swait(NCHUNK - 4 + b, b)

    plsc.subcore_barrier()
    obase = c * N_PAD + s * ROWS_PER_TILE
    pltpu.sync_copy(agg_sh.at[pl.ds(base, ROWS_PER_TILE)],
                    agg_out.at[pl.ds(obase, ROWS_PER_TILE)])
    pltpu.sync_copy(deg_sh.at[pl.ds(base, ROWS_PER_TILE)],
                    deg_out.at[pl.ds(obase, ROWS_PER_TILE)])


_sc_agg_deg = pl.kernel(
    _sc_body,
    out_type=(
        jax.ShapeDtypeStruct((NC * N_PAD, DH), jnp.bfloat16),
        jax.ShapeDtypeStruct((NC * N_PAD, DEG_W), jnp.float32),
    ),
    mesh=plsc.VectorSubcoreMesh(core_axis_name="c", subcore_axis_name="s"),
    compiler_params=pltpu.CompilerParams(use_tc_tiling_on_sc=False),
    scratch_types=[
        pltpu.VMEM((NCHUNK, CHUNK), jnp.int32),
        pltpu.VMEM((NCHUNK, CHUNK), jnp.int32),
        pltpu.VMEM((4, CHUNK, DH), jnp.bfloat16),
        pltpu.VMEM((CHUNK, DEG_W), jnp.float32),
        pltpu.VMEM((ZROWS, DH), jnp.bfloat16),
        pltpu.VMEM((ZROWS, DEG_W), jnp.float32),
        pltpu.VMEM_SHARED((N_PAD, DH), jnp.bfloat16),
        pltpu.VMEM_SHARED((N_PAD, DEG_W), jnp.float32),
        pltpu.SemaphoreType.DMA((4,)),
        pltpu.SemaphoreType.DMA((4,)),
    ],
)

BLK = 2000
NBLK = N_NODES // BLK


def _make_tc_layer(emit_table):
    def body(h_ref, a0_ref, a1_ref, d0_ref, d1_ref, ws_ref, wn_ref, b_ref,
             *o_refs):
        agg = jnp.concatenate([a0_ref[0], a1_ref[0]],
                              axis=1).astype(jnp.float32)
        deg = d0_ref[0, :, 0:1] + d1_ref[0, :, 0:1]
        mean = agg / jnp.maximum(deg, 1.0)
        out = (jnp.dot(h_ref[...], ws_ref[...],
                       preferred_element_type=jnp.float32)
               + jnp.dot(mean, wn_ref[...],
                         preferred_element_type=jnp.float32)
               + b_ref[...])
        if emit_table:
            out = jnp.maximum(out, 0.0)
        o_refs[0][...] = out

    return pl.pallas_call(
        body,
        grid=(NBLK,),
        in_specs=[
            pl.BlockSpec((BLK, D), lambda i: (i, 0)),
            pl.BlockSpec((1, BLK, DH), lambda i: (0, i, 0)),
            pl.BlockSpec((1, BLK, DH), lambda i: (1, i, 0)),
            pl.BlockSpec((1, BLK, DEG_W), lambda i: (0, i, 0)),
            pl.BlockSpec((1, BLK, DEG_W), lambda i: (1, i, 0)),
            pl.BlockSpec((D, D), lambda i: (0, 0)),
            pl.BlockSpec((D, D), lambda i: (0, 0)),
            pl.BlockSpec((1, D), lambda i: (0, 0)),
        ],
        out_specs=[pl.BlockSpec((BLK, D), lambda i: (i, 0))],
        out_shape=[jax.ShapeDtypeStruct((N_NODES, D), jnp.float32)],
    )


_tc_layer1 = _make_tc_layer(True)
_tc_layer2 = _make_tc_layer(False)


def _split_cols(h):
    return jnp.concatenate([h[:, :DH], h[:, DH:]], axis=0).astype(jnp.bfloat16)


def kernel(x, edge_index, W_self1, W_neigh1, b1, W_self2, W_neigh2, b2):
    ei = edge_index.astype(jnp.int32)
    e3 = ei.reshape(2, NS, N_EDGES // NS)
    npad = EDGES_PER_TILE - N_EDGES // NS
    src_p = jnp.pad(e3[0], ((0, 0), (0, npad)))
    dst_p = jnp.pad(e3[1], ((0, 0), (0, npad)), constant_values=N_NODES)
    src_r = src_p.reshape(1, NS, NCHUNK, CHUNK)
    dst_r = dst_p.reshape(1, NS, NCHUNK, CHUNK)
    src = jnp.concatenate([src_r, src_r + N_NODES],
                          axis=0).reshape(NW, NCHUNK, CHUNK)
    dst = jnp.concatenate([dst_r, dst_r], axis=0).reshape(NW, NCHUNK, CHUNK)

    agg1, deg = _sc_agg_deg(_split_cols(x), src, dst)
    agg1 = agg1.reshape(NC, N_PAD, DH)
    deg = deg.reshape(NC, N_PAD, DEG_W)
    (h1,) = _tc_layer1(x, agg1, agg1, deg, deg, W_self1, W_neigh1,
                       b1.reshape(1, D))

    agg2, _ = _sc_agg_deg(_split_cols(h1), src, dst)
    agg2 = agg2.reshape(NC, N_PAD, DH)
    (out,) = _tc_layer2(h1, agg2, agg2, deg, deg, W_self2, W_neigh2,
                        b2.reshape(1, D))
    return out


# speedup vs baseline: 1.7458x; 1.7458x over previous
import jax
import jax.numpy as jnp
from jax import lax
from jax.experimental import pallas as pl
from jax.experimental.pallas import tpu as pltpu
from jax.experimental.pallas import tpu_sc as plsc

N_NODES = 10000
N_EDGES = 320000
D = 128
DH = D // 2

NC = 2
NS = 16
NW = NC * NS

EDGES_PER_TILE = N_EDGES // NS
CHUNK = 125
NCHUNK = EDGES_PER_TILE // CHUNK
N_PAD = 10240
ROWS_PER_TILE = N_PAD // NS
ZROWS = 32
DEG_W = 16


def _sc_body(h_hbm, src_hbm, dst_hbm, agg_out, deg_out, src_v, dst_v, rows,
             ones_v, zb, zbd, agg_sh, deg_sh, gsem, ssem):
    c = lax.axis_index("c")
    s = lax.axis_index("s")
    wid = c * NS + s

    cp_src = pltpu.async_copy(src_hbm.at[wid], src_v, gsem.at[0])
    cp_dst = pltpu.async_copy(dst_hbm.at[wid], dst_v, gsem.at[1])

    zeros32 = jnp.zeros((32,), jnp.bfloat16)
    zeros16 = jnp.zeros((16,), jnp.float32)
    ones16 = jnp.ones((16,), jnp.float32)

    def zrow(i, carry):
        for k in range(DH // 32):
            zb[i, pl.ds(k * 32, 32)] = zeros32
        zbd[i] = zeros16
        return carry

    lax.fori_loop(0, ZROWS, zrow, 0)

    def orow(i, carry):
        ones_v[i] = ones16
        return carry

    lax.fori_loop(0, CHUNK, orow, 0)

    base = s * ROWS_PER_TILE
    zcps = []
    for k in range(ROWS_PER_TILE // ZROWS):
        zcps.append(pltpu.async_copy(
            zb, agg_sh.at[pl.ds(base + k * ZROWS, ZROWS)], ssem.at[k % 4]))
        zcps.append(pltpu.async_copy(
            zbd, deg_sh.at[pl.ds(base + k * ZROWS, ZROWS)], ssem.at[k % 4]))
    for cp in zcps:
        cp.wait()
    cp_src.wait()
    cp_dst.wait()
    plsc.subcore_barrier()

    def gstart(j, b):
        pltpu.async_copy(h_hbm.at[src_v.at[j]], rows.at[b], gsem.at[b])

    def gwait(j, b):
        pltpu.make_async_copy(h_hbm.at[src_v.at[j]], rows.at[b],
                              gsem.at[b]).wait()

    def sstart(j, b):
        pltpu.async_copy(rows.at[b], agg_sh.at[dst_v.at[j]], ssem.at[b],
                         add=True)

    def swait(j, b):
        pltpu.make_async_copy(rows.at[b], agg_sh.at[dst_v.at[j]],
                              ssem.at[b]).wait()

    def dscat(j, deg_core):
        @pl.when(c == deg_core)
        def _():
            pltpu.sync_copy(ones_v, deg_sh.at[dst_v.at[j]], add=True)

    gstart(0, 0)
    gstart(1, 1)
    gstart(2, 2)

    def step(k4, carry):
        for b in range(4):
            j = 4 * k4 + b
            gwait(j, b)
            sstart(j, b)
            dscat(j, b % 2)
            nb = (b + 3) % 4
            if b == 0:
                @pl.when(k4 > 0)
                def _():
                    swait(j - 1, nb)
                gstart(j + 3, nb)
            else:
                @pl.when(k4 < NCHUNK // 4 - 1)
                def _():
                    swait(j - 1, nb)
                    gstart(j + 3, nb)
        return carry

    lax.fori_loop(0, NCHUNK // 4, step, 0)
    for b in range(4):
        swait(NCHUNK - 4 + b, b)

    plsc.subcore_barrier()
    obase = c * N_PAD + s * ROWS_PER_TILE
    pltpu.sync_copy(agg_sh.at[pl.ds(base, ROWS_PER_TILE)],
                    agg_out.at[pl.ds(obase, ROWS_PER_TILE)])
    pltpu.sync_copy(deg_sh.at[pl.ds(base, ROWS_PER_TILE)],
                    deg_out.at[pl.ds(obase, ROWS_PER_TILE)])


_sc_agg_deg = pl.kernel(
    _sc_body,
    out_type=(
        jax.ShapeDtypeStruct((NC * N_PAD, DH), jnp.bfloat16),
        jax.ShapeDtypeStruct((NC * N_PAD, DEG_W), jnp.float32),
    ),
    mesh=plsc.VectorSubcoreMesh(core_axis_name="c", subcore_axis_name="s"),
    compiler_params=pltpu.CompilerParams(use_tc_tiling_on_sc=False),
    scratch_types=[
        pltpu.VMEM((NCHUNK, CHUNK), jnp.int32),
        pltpu.VMEM((NCHUNK, CHUNK), jnp.int32),
        pltpu.VMEM((4, CHUNK, DH), jnp.bfloat16),
        pltpu.VMEM((CHUNK, DEG_W), jnp.float32),
        pltpu.VMEM((ZROWS, DH), jnp.bfloat16),
        pltpu.VMEM((ZROWS, DEG_W), jnp.float32),
        pltpu.VMEM_SHARED((N_PAD, DH), jnp.bfloat16),
        pltpu.VMEM_SHARED((N_PAD, DEG_W), jnp.float32),
        pltpu.SemaphoreType.DMA((4,)),
        pltpu.SemaphoreType.DMA((4,)),
    ],
)

BLK = 2000
NBLK = N_NODES // BLK


def _make_tc_layer(emit_table):
    def body(h_ref, a0_ref, a1_ref, d0_ref, d1_ref, ws_ref, wn_ref, b_ref,
             *o_refs):
        agg = jnp.concatenate([a0_ref[0], a1_ref[0]],
                              axis=1).astype(jnp.float32)
        deg = d0_ref[0, :, 0:1] + d1_ref[0, :, 0:1]
        mean = agg / jnp.maximum(deg, 1.0)
        out = (jnp.dot(h_ref[...], ws_ref[...],
                       preferred_element_type=jnp.float32)
               + jnp.dot(mean, wn_ref[...],
                         preferred_element_type=jnp.float32)
               + b_ref[...])
        if emit_table:
            out = jnp.maximum(out, 0.0)
            o_refs[1][0] = out[:, :DH].astype(jnp.bfloat16)
            o_refs[1][1] = out[:, DH:].astype(jnp.bfloat16)
        o_refs[0][...] = out

    return pl.pallas_call(
        body,
        grid=(NBLK,),
        in_specs=[
            pl.BlockSpec((BLK, D), lambda i: (i, 0)),
            pl.BlockSpec((1, BLK, DH), lambda i: (0, i, 0)),
            pl.BlockSpec((1, BLK, DH), lambda i: (1, i, 0)),
            pl.BlockSpec((1, BLK, DEG_W), lambda i: (0, i, 0)),
            pl.BlockSpec((1, BLK, DEG_W), lambda i: (1, i, 0)),
            pl.BlockSpec((D, D), lambda i: (0, 0)),
            pl.BlockSpec((D, D), lambda i: (0, 0)),
            pl.BlockSpec((1, D), lambda i: (0, 0)),
        ],
        out_specs=(
            [pl.BlockSpec((BLK, D), lambda i: (i, 0))]
            + ([pl.BlockSpec((2, BLK, DH), lambda i: (0, i, 0))]
               if emit_table else [])
        ),
        out_shape=(
            [jax.ShapeDtypeStruct((N_NODES, D), jnp.float32)]
            + ([jax.ShapeDtypeStruct((2, N_NODES, DH), jnp.bfloat16)]
               if emit_table else [])
        ),
    )


_tc_layer1 = _make_tc_layer(True)
_tc_layer2 = _make_tc_layer(False)


def _split_body(h_ref, o_ref):
    o_ref[0] = h_ref[:, :DH].astype(jnp.bfloat16)
    o_ref[1] = h_ref[:, DH:].astype(jnp.bfloat16)


_split_k = pl.pallas_call(
    _split_body,
    grid=(NBLK,),
    in_specs=[pl.BlockSpec((BLK, D), lambda i: (i, 0))],
    out_specs=pl.BlockSpec((2, BLK, DH), lambda i: (0, i, 0)),
    out_shape=jax.ShapeDtypeStruct((2, N_NODES, DH), jnp.bfloat16),
)


def kernel(x, edge_index, W_self1, W_neigh1, b1, W_self2, W_neigh2, b2):
    ei = edge_index.astype(jnp.int32)
    src_r = ei[0].reshape(1, NS, NCHUNK, CHUNK)
    dst_r = ei[1].reshape(1, NS, NCHUNK, CHUNK)
    src = jnp.concatenate([src_r, src_r + N_NODES],
                          axis=0).reshape(NW, NCHUNK, CHUNK)
    dst = jnp.concatenate([dst_r, dst_r], axis=0).reshape(NW, NCHUNK, CHUNK)

    agg1, deg = _sc_agg_deg(_split_k(x).reshape(NC * N_NODES, DH), src, dst)
    agg1 = agg1.reshape(NC, N_PAD, DH)
    deg = deg.reshape(NC, N_PAD, DEG_W)
    h1, tab2 = _tc_layer1(x, agg1, agg1, deg, deg, W_self1, W_neigh1,
                          b1.reshape(1, D))

    agg2, _ = _sc_agg_deg(tab2.reshape(NC * N_NODES, DH), src, dst)
    agg2 = agg2.reshape(NC, N_PAD, DH)
    (out,) = _tc_layer2(h1, agg2, agg2, deg, deg, W_self2, W_neigh2,
                        b2.reshape(1, D))
    return out


# speedup vs baseline: 1.8153x; 1.0398x over previous
import jax
import jax.numpy as jnp
from jax import lax
from jax.experimental import pallas as pl
from jax.experimental.pallas import tpu as pltpu
from jax.experimental.pallas import tpu_sc as plsc

N_NODES = 10000
N_EDGES = 320000
D = 128
DH = D // 2

NC = 2
NS = 16
NW = NC * NS

EDGES_PER_TILE = N_EDGES // NS
CHUNK = 125
NCHUNK = EDGES_PER_TILE // CHUNK
N_PAD = 10240
ROWS_PER_TILE = N_PAD // NS
ZROWS = 32
DEG_W = 16


def _sc_body(h_hbm, src_hbm, dst_hbm, agg_out, deg_out, src_v, dst_v, rows,
             ones_v, zb, zbd, agg_sh, deg_sh, gsem, ssem, dsem):
    c = lax.axis_index("c")
    s = lax.axis_index("s")
    wid = c * NS + s

    cp_src = pltpu.async_copy(src_hbm.at[wid], src_v, gsem.at[0])
    cp_dst = pltpu.async_copy(dst_hbm.at[s], dst_v, gsem.at[1])

    zeros32 = jnp.zeros((32,), jnp.bfloat16)
    zeros16 = jnp.zeros((16,), jnp.float32)
    ones16 = jnp.ones((16,), jnp.float32)

    def zrow(i, carry):
        for k in range(DH // 32):
            zb[i, pl.ds(k * 32, 32)] = zeros32
        zbd[i] = zeros16
        return carry

    lax.fori_loop(0, ZROWS, zrow, 0)

    def orow(i, carry):
        ones_v[i] = ones16
        return carry

    lax.fori_loop(0, CHUNK, orow, 0)

    base = s * ROWS_PER_TILE
    zcps = []
    for k in range(ROWS_PER_TILE // ZROWS):
        zcps.append(pltpu.async_copy(
            zb, agg_sh.at[pl.ds(base + k * ZROWS, ZROWS)], ssem.at[k % 4]))
        zcps.append(pltpu.async_copy(
            zbd, deg_sh.at[pl.ds(base + k * ZROWS, ZROWS)], ssem.at[k % 4]))
    for cp in zcps:
        cp.wait()
    cp_src.wait()
    cp_dst.wait()
    plsc.subcore_barrier()

    def gstart(j, b):
        pltpu.async_copy(h_hbm.at[src_v.at[j]], rows.at[b], gsem.at[b])

    def gwait(j, b):
        pltpu.make_async_copy(h_hbm.at[src_v.at[j]], rows.at[b],
                              gsem.at[b]).wait()

    def sstart(j, b):
        pltpu.async_copy(rows.at[b], agg_sh.at[dst_v.at[j]], ssem.at[b],
                         add=True)

    def swait(j, b):
        pltpu.make_async_copy(rows.at[b], agg_sh.at[dst_v.at[j]],
                              ssem.at[b]).wait()

    def dscat(j, b):
        @pl.when(c == b % 2)
        def _():
            pltpu.async_copy(ones_v, deg_sh.at[dst_v.at[j]], dsem.at[b],
                             add=True)

    def dswait(j, b):
        @pl.when(c == b % 2)
        def _():
            pltpu.make_async_copy(ones_v, deg_sh.at[dst_v.at[j]],
                                  dsem.at[b]).wait()

    gstart(0, 0)
    gstart(1, 1)
    gstart(2, 2)

    def step(k4, carry):
        for b in range(4):
            j = 4 * k4 + b
            gwait(j, b)
            sstart(j, b)

            @pl.when(k4 > 0)
            def _():
                dswait(j - 4, b)

            dscat(j, b)
            nb = (b + 3) % 4
            if b == 0:
                @pl.when(k4 > 0)
                def _():
                    swait(j - 1, nb)
                gstart(j + 3, nb)
            else:
                @pl.when(k4 < NCHUNK // 4 - 1)
                def _():
                    swait(j - 1, nb)
                    gstart(j + 3, nb)
        return carry

    lax.fori_loop(0, NCHUNK // 4, step, 0)
    for b in range(4):
        swait(NCHUNK - 4 + b, b)
        dswait(NCHUNK - 4 + b, b)

    plsc.subcore_barrier()
    obase = c * N_PAD + s * ROWS_PER_TILE
    pltpu.sync_copy(agg_sh.at[pl.ds(base, ROWS_PER_TILE)],
                    agg_out.at[pl.ds(obase, ROWS_PER_TILE)])
    pltpu.sync_copy(deg_sh.at[pl.ds(base, ROWS_PER_TILE)],
                    deg_out.at[pl.ds(obase, ROWS_PER_TILE)])


_sc_agg_deg = pl.kernel(
    _sc_body,
    out_type=(
        jax.ShapeDtypeStruct((NC * N_PAD, DH), jnp.bfloat16),
        jax.ShapeDtypeStruct((NC * N_PAD, DEG_W), jnp.float32),
    ),
    mesh=plsc.VectorSubcoreMesh(core_axis_name="c", subcore_axis_name="s"),
    compiler_params=pltpu.CompilerParams(use_tc_tiling_on_sc=False),
    scratch_types=[
        pltpu.VMEM((NCHUNK, CHUNK), jnp.int32),
        pltpu.VMEM((NCHUNK, CHUNK), jnp.int32),
        pltpu.VMEM((4, CHUNK, DH), jnp.bfloat16),
        pltpu.VMEM((CHUNK, DEG_W), jnp.float32),
        pltpu.VMEM((ZROWS, DH), jnp.bfloat16),
        pltpu.VMEM((ZROWS, DEG_W), jnp.float32),
        pltpu.VMEM_SHARED((N_PAD, DH), jnp.bfloat16),
        pltpu.VMEM_SHARED((N_PAD, DEG_W), jnp.float32),
        pltpu.SemaphoreType.DMA((4,)),
        pltpu.SemaphoreType.DMA((4,)),
        pltpu.SemaphoreType.DMA((4,)),
    ],
)

BLK = 2000
NBLK = N_NODES // BLK


def _make_tc_layer(emit_table):
    def body(h_ref, a0_ref, a1_ref, d0_ref, d1_ref, ws_ref, wn_ref, b_ref,
             *o_refs):
        agg = jnp.concatenate([a0_ref[0], a1_ref[0]],
                              axis=1).astype(jnp.float32)
        deg = d0_ref[0, :, 0:1] + d1_ref[0, :, 0:1]
        mean = agg / jnp.maximum(deg, 1.0)
        out = (jnp.dot(h_ref[...], ws_ref[...],
                       preferred_element_type=jnp.float32)
               + jnp.dot(mean, wn_ref[...],
                         preferred_element_type=jnp.float32)
               + b_ref[...])
        if emit_table:
            out = jnp.maximum(out, 0.0)
            o_refs[1][0] = out[:, :DH].astype(jnp.bfloat16)
            o_refs[1][1] = out[:, DH:].astype(jnp.bfloat16)
        o_refs[0][...] = out

    return pl.pallas_call(
        body,
        grid=(NBLK,),
        in_specs=[
            pl.BlockSpec((BLK, D), lambda i: (i, 0)),
            pl.BlockSpec((1, BLK, DH), lambda i: (0, i, 0)),
            pl.BlockSpec((1, BLK, DH), lambda i: (1, i, 0)),
            pl.BlockSpec((1, BLK, DEG_W), lambda i: (0, i, 0)),
            pl.BlockSpec((1, BLK, DEG_W), lambda i: (1, i, 0)),
            pl.BlockSpec((D, D), lambda i: (0, 0)),
            pl.BlockSpec((D, D), lambda i: (0, 0)),
            pl.BlockSpec((1, D), lambda i: (0, 0)),
        ],
        out_specs=(
            [pl.BlockSpec((BLK, D), lambda i: (i, 0))]
            + ([pl.BlockSpec((2, BLK, DH), lambda i: (0, i, 0))]
               if emit_table else [])
        ),
        out_shape=(
            [jax.ShapeDtypeStruct((N_NODES, D), jnp.float32)]
            + ([jax.ShapeDtypeStruct((2, N_NODES, DH), jnp.bfloat16)]
               if emit_table else [])
        ),
    )


_tc_layer1 = _make_tc_layer(True)
_tc_layer2 = _make_tc_layer(False)


def _split_body(h_ref, o_ref):
    o_ref[0] = h_ref[:, :DH].astype(jnp.bfloat16)
    o_ref[1] = h_ref[:, DH:].astype(jnp.bfloat16)


_split_k = pl.pallas_call(
    _split_body,
    grid=(NBLK,),
    in_specs=[pl.BlockSpec((BLK, D), lambda i: (i, 0))],
    out_specs=pl.BlockSpec((2, BLK, DH), lambda i: (0, i, 0)),
    out_shape=jax.ShapeDtypeStruct((2, N_NODES, DH), jnp.bfloat16),
)


def kernel(x, edge_index, W_self1, W_neigh1, b1, W_self2, W_neigh2, b2):
    ei = edge_index.astype(jnp.int32)
    src_r = ei[0].reshape(1, NS, NCHUNK, CHUNK)
    dst_r = ei[1].reshape(1, NS, NCHUNK, CHUNK)
    src = jnp.concatenate([src_r, src_r + N_NODES],
                          axis=0).reshape(NW, NCHUNK, CHUNK)
    dst = dst_r.reshape(NS, NCHUNK, CHUNK)

    agg1, deg = _sc_agg_deg(_split_k(x).reshape(NC * N_NODES, DH), src, dst)
    agg1 = agg1.reshape(NC, N_PAD, DH)
    deg = deg.reshape(NC, N_PAD, DEG_W)
    h1, tab2 = _tc_layer1(x, agg1, agg1, deg, deg, W_self1, W_neigh1,
                          b1.reshape(1, D))

    agg2, _ = _sc_agg_deg(tab2.reshape(NC * N_NODES, DH), src, dst)
    agg2 = agg2.reshape(NC, N_PAD, DH)
    (out,) = _tc_layer2(h1, agg2, agg2, deg, deg, W_self2, W_neigh2,
                        b2.reshape(1, D))
    return out
